# baseline (speedup 1.0000x reference)
import functools
import math

import jax
import jax.numpy as jnp
from jax import lax
from jax.experimental import pallas as pl
from jax.experimental.pallas import tpu as pltpu
from jax.experimental.pallas import tpu_sc as plsc

_KNN = 5
_S = 5
_SS = _S * _S
_CENTER = (_SS - 1) // 2
_SIGMA = 1.0
_CUTOFF = 1.0
_NCLS = 20
_KROWS = 32
_UNP_ROW = 25
_SROWS = 8

_NC, _NS = 2, 16
_NW = _NC * _NS
_NBUF = 3


def _inv_gauss_weights():
    x = jnp.arange(_S)
    x_grid = jnp.tile(x, _S).reshape(_S, _S)
    y_grid = x_grid.T
    mean = (_S - 1) / 2.0
    var = _SIGMA ** 2.0
    g = (1.0 / (2.0 * math.pi * var)) * jnp.exp(
        -((x_grid - mean) ** 2.0 + (y_grid - mean) ** 2.0) / (2.0 * var))
    g = g / jnp.sum(g)
    w = (1.0 - g).reshape(_SS).astype(jnp.float32)
    return jnp.concatenate([w, jnp.zeros((_KROWS - _SS,), jnp.float32)])


def _pipelined_gather(table_hbm, idx_hbm, out_hbm, bufs, ks, n_points, pt,
                      base):
    idx_v = bufs[0:_NBUF]
    buf_v = bufs[_NBUF:2 * _NBUF]
    semi = bufs[2 * _NBUF:3 * _NBUF]
    semg = bufs[3 * _NBUF:4 * _NBUF]
    sems = bufs[4 * _NBUF:5 * _NBUF]

    def row(k):
        return pl.ds(k * n_points + base, pt)

    nk = len(ks)
    ld = [None] * nk
    gat = [None] * nk
    st = [None] * nk

    def fire_st(i):
        s = i % _NBUF
        gat[i].wait()
        st[i] = pltpu.async_copy(buf_v[s], out_hbm.at[row(ks[i])], sems[s])

    ld[0] = pltpu.async_copy(idx_hbm.at[row(ks[0])], idx_v[0], semi[0])
    for i in range(nk):
        s = i % _NBUF
        ld[i].wait()
        if i >= _NBUF:
            st[i - _NBUF].wait()
        gat[i] = pltpu.async_copy(table_hbm.at[idx_v[s]], buf_v[s], semg[s])
        if i + 1 < nk:
            if i + 1 >= _NBUF:
                fire_st(i + 1 - _NBUF)
            ld[i + 1] = pltpu.async_copy(
                idx_hbm.at[row(ks[i + 1])], idx_v[(i + 1) % _NBUF],
                semi[(i + 1) % _NBUF])
    for i in range(max(0, nk - _NBUF), nk):
        fire_st(i)
        st[i].wait()


_CHUNK = 128
_ROWBLK = 16


def _sc_gather_range(j0, o, rng_tbl, unproj, n_points, wrow):
    pt = n_points // _NW
    nch = pt // _CHUNK
    cb = _KROWS * _CHUNK
    rows5 = _S * _CHUNK
    mesh = plsc.VectorSubcoreMesh(core_axis_name="c", subcore_axis_name="s",
                                  num_cores=_NC, num_subcores=_NS)
    scratch = (
        [pltpu.VMEM((pt,), jnp.int32),
         pltpu.VMEM((pt,), jnp.int32),
         pltpu.VMEM((pt,), jnp.float32)]
        + [pltpu.VMEM((rows5,), jnp.int32) for _ in range(4)]
        + [pltpu.VMEM((2, rows5, _ROWBLK), jnp.float32) for _ in range(2)]
        + [pltpu.VMEM((cb,), jnp.float32) for _ in range(2)]
        + [pltpu.SemaphoreType.DMA for _ in range(5)]
    )

    @functools.partial(
        pl.kernel,
        out_type=jax.ShapeDtypeStruct((_NW * nch * cb,), jnp.float32),
        mesh=mesh,
        scratch_types=scratch,
        compiler_params=pltpu.CompilerParams(needs_layout_passes=False,
                                             use_tc_tiling_on_sc=False),
    )
    def sc1(j0_hbm, o_hbm, tbl_hbm, unp_hbm, stage_hbm, *bufs):
        j0_v, o_v, unp_v = bufs[0:3]
        ja = bufs[3:5]
        jb = bufs[5:7]
        rb = bufs[7:9]
        ob = bufs[9:11]
        semg = bufs[11:13]
        semst = bufs[13:15]
        semin = bufs[15]
        roff = rows5 * _ROWBLK
        wid = lax.axis_index("s") * _NC + lax.axis_index("c")
        base = wid * pt
        h1 = pltpu.async_copy(j0_hbm.at[pl.ds(base, pt)], j0_v, semin)
        h2 = pltpu.async_copy(o_hbm.at[pl.ds(base, pt)], o_v, semin)
        h3 = pltpu.async_copy(unp_hbm.at[pl.ds(base, pt)], unp_v, semin)
        h1.wait()
        h2.wait()
        h3.wait()
        iota = lax.iota(jnp.int32, 16)

        def build_j(c, s):
            for g in range(_CHUNK // 16):
                j0g = j0_v[pl.ds(c * _CHUNK + g * 16, 16)]
                for dy in range(_S):
                    jv = j0g + dy * wrow
                    ja[s][pl.ds(dy * _CHUNK + g * 16, 16)] = jv
                    jb[s][pl.ds(dy * _CHUNK + g * 16, 16)] = jv + 1

        def fire_gathers(s):
            pltpu.async_copy(tbl_hbm.at[ja[s]], rb[s].at[0], semg[s])
            pltpu.async_copy(tbl_hbm.at[jb[s]], rb[s].at[1], semg[s])

        def wait_gathers(s):
            pltpu.make_async_copy(tbl_hbm.at[ja[s]], rb[s].at[0],
                                  semg[s]).wait()
            pltpu.make_async_copy(tbl_hbm.at[jb[s]], rb[s].at[1],
                                  semg[s]).wait()

        def extract(c, s):
            for g in range(_CHUNK // 16):
                gof = g * 16
                og = o_v[pl.ds(c * _CHUNK + gof, 16)]
                ung = unp_v[pl.ds(c * _CHUNK + gof, 16)]
                ob[s][pl.ds(_CENTER * _CHUNK + gof, 16)] = ung
                ob[s][pl.ds(_UNP_ROW * _CHUNK + gof, 16)] = ung
                ts = [og + dx for dx in range(_S)]
                cols = [t & 15 for t in ts]
                sels = [t >> 4 for t in ts]
                for dy in range(_S):
                    row = iota + (dy * _CHUNK + gof)
                    for dx in range(_S):
                        k = dy * _S + dx
                        if k == _CENTER:
                            continue
                        v = plsc.load_gather(rb[s], [sels[dx], row, cols[dx]])
                        ob[s][pl.ds(k * _CHUNK + gof, 16)] = v

        def st_slice(c):
            return stage_hbm.at[pl.ds((wid * nch + c) * cb, cb)]

        def fire_store(c, s):
            pltpu.async_copy(ob[s], st_slice(c), semst[s])

        def wait_store(c, s):
            pltpu.make_async_copy(ob[s], st_slice(c), semst[s]).wait()

        def body(cc, _):
            for sub in range(2):
                c = cc * 2 + sub
                s = sub

                @pl.when(c >= 2)
                def _():
                    wait_store(c - 2, s)

                build_j(c, s)
                fire_gathers(s)

                @pl.when(c >= 1)
                def _():
                    o2 = 1 - s
                    wait_gathers(o2)
                    extract(c - 1, o2)
                    fire_store(c - 1, o2)

            return 0

        lax.fori_loop(0, nch // 2, body, 0)
        wait_gathers(1)
        extract(nch - 1, 1)
        fire_store(nch - 1, 1)
        wait_store(nch - 2, 0)
        wait_store(nch - 1, 1)

    return sc1(j0, o, rng_tbl, unproj)


def _sc_gather_cls(sel_idx, cls_pad, n_points):
    pt = n_points // _NW
    mesh = plsc.VectorSubcoreMesh(core_axis_name="c", subcore_axis_name="s",
                                  num_cores=_NC, num_subcores=_NS)
    scratch = ([pltpu.VMEM((pt,), jnp.int32) for _ in range(_NBUF)]
               + [pltpu.VMEM((pt,), jnp.int32) for _ in range(_NBUF)]
               + [pltpu.SemaphoreType.DMA for _ in range(3 * _NBUF)])

    @functools.partial(
        pl.kernel,
        out_type=jax.ShapeDtypeStruct((_SROWS * n_points,), jnp.int32),
        mesh=mesh,
        scratch_types=scratch,
    )
    def sc2(selidx_hbm, cls_hbm, cls5_hbm, *bufs):
        wid = lax.axis_index("s") * _NC + lax.axis_index("c")
        base = wid * pt
        _pipelined_gather(cls_hbm, selidx_hbm, cls5_hbm, bufs,
                          list(range(_KNN)), n_points, pt, base)

    return sc2(sel_idx, cls_pad)


def _tc_select_body(sentinel, grng_ref, base_ref, w_ref, offs_ref, o_ref):
    g = grng_ref[...]
    w = w_ref[...]
    offs = offs_ref[...]
    b = g.shape[1]
    base = base_ref[...].reshape(1, b)
    r = g[_UNP_ROW:_UNP_ROW + 1, :]
    rows = lax.broadcasted_iota(jnp.int32, (_KROWS, b), 0)
    d = jnp.abs(g - r) * w
    d = jnp.where(rows < _SS, d, jnp.inf)

    sel = []
    for _ in range(_KNN):
        m = jnp.min(d, axis=0, keepdims=True)
        ki = jnp.min(jnp.where(d == m, rows, _KROWS), axis=0, keepdims=True)
        hit = rows == ki
        off = jnp.max(jnp.where(hit, offs, -1), axis=0, keepdims=True)
        flat = jnp.where(m > _CUTOFF, sentinel, base + off)
        sel.append(flat)
        d = jnp.where(hit, jnp.inf, d)
    zero = jnp.zeros_like(sel[0])
    o_ref[...] = jnp.concatenate(sel + [zero] * (_SROWS - _KNN), axis=0)


def _tc_select(g_stage, base3, w_col, offs_col, sentinel, n_points):
    nb = n_points // _CHUNK
    return pl.pallas_call(
        functools.partial(_tc_select_body, sentinel),
        grid=(nb,),
        in_specs=[
            pl.BlockSpec((_KROWS, _CHUNK), lambda i: (i, 0)),
            pl.BlockSpec((1, 1, _CHUNK), lambda i: (i, 0, 0)),
            pl.BlockSpec((_KROWS, 1), lambda i: (0, 0)),
            pl.BlockSpec((_KROWS, 1), lambda i: (0, 0)),
        ],
        out_specs=pl.BlockSpec((_SROWS, _CHUNK), lambda i: (0, i)),
        out_shape=jax.ShapeDtypeStruct((_SROWS, n_points), jnp.int32),
    )(g_stage, base3, w_col, offs_col)


def _tc_vote_body(cls_ref, o_ref):
    cl = cls_ref[...]
    b = cl.shape[1]
    sel = [cl[i:i + 1, :] for i in range(_KNN)]
    ones = jnp.ones_like(sel[0])
    cnt = [ones] * _KNN
    for i in range(_KNN):
        for j in range(i + 1, _KNN):
            e = (sel[i] == sel[j]).astype(jnp.int32)
            cnt[i] = cnt[i] + e
            cnt[j] = cnt[j] + e
    neg = jnp.full_like(ones, -1000)
    key = neg
    for i in range(_KNN):
        c = sel[i]
        valid = (c >= 1) & (c < _NCLS)
        key = jnp.maximum(key, jnp.where(valid, cnt[i] * 32 - c, neg))
    best = jnp.where(key == -1000, 1, 32 - (key & 31))
    o_ref[...] = best.reshape(1, 1, b)


def _tc_vote(cls5, n_points, block=2048):
    nb = n_points // block
    return pl.pallas_call(
        _tc_vote_body,
        grid=(nb,),
        in_specs=[pl.BlockSpec((_SROWS, block), lambda i: (0, i))],
        out_specs=pl.BlockSpec((1, 1, block), lambda i: (i, 0, 0)),
        out_shape=jax.ShapeDtypeStruct((nb, 1, block), jnp.int32),
    )(cls5)


def kernel(proj_range, unproj_range, proj_argmax, px, py):
    h, w = proj_range.shape
    p = unproj_range.shape[0]
    pad = (_S - 1) // 2
    wp = w + 2 * pad
    wrow = (w + 16) // _ROWBLK
    rng_tbl = jnp.pad(proj_range, ((pad, pad + 1), (pad, 14))).reshape(
        -1, _ROWBLK)
    j0 = py * wrow + (px >> 4)
    o = px & 15
    cls_flat = jnp.pad(proj_argmax, pad).reshape(-1)
    npix = cls_flat.shape[0]
    cls_pad = jnp.concatenate([cls_flat, jnp.full((8,), _NCLS, jnp.int32)])
    sentinel = npix
    base = py * wp + px
    offs = [dy * wp + dx for dy in range(_S) for dx in range(_S)]
    g_stage = _sc_gather_range(j0, o, rng_tbl, unproj_range, p, wrow)
    g_stage = g_stage.reshape((p // _CHUNK) * _KROWS, _CHUNK)
    w_col = _inv_gauss_weights().reshape(_KROWS, 1)
    offs_col = jnp.array(offs + [0] * (_KROWS - _SS),
                         jnp.int32).reshape(_KROWS, 1)
    base3 = base.reshape(p // _CHUNK, 1, _CHUNK)
    sel_idx = _tc_select(g_stage, base3, w_col, offs_col, sentinel, p)
    cls5 = _sc_gather_cls(sel_idx.reshape(-1), cls_pad, p)
    out3 = _tc_vote(cls5.reshape(_SROWS, p), p)
    return out3.reshape(p)


# speedup vs baseline: 2.0279x; 2.0279x over previous
import functools
import math

import jax
import jax.numpy as jnp
from jax import lax
from jax.experimental import pallas as pl
from jax.experimental.pallas import tpu as pltpu
from jax.experimental.pallas import tpu_sc as plsc

_KNN = 5
_S = 5
_SS = _S * _S
_CENTER = (_SS - 1) // 2
_SIGMA = 1.0
_CUTOFF = 1.0
_NCLS = 20
_KROWS = 32
_UNP_ROW = 25
_SROWS = 8

_NC, _NS = 2, 16
_NW = _NC * _NS
_NBUF = 3


def _inv_gauss_weights():
    x = jnp.arange(_S)
    x_grid = jnp.tile(x, _S).reshape(_S, _S)
    y_grid = x_grid.T
    mean = (_S - 1) / 2.0
    var = _SIGMA ** 2.0
    g = (1.0 / (2.0 * math.pi * var)) * jnp.exp(
        -((x_grid - mean) ** 2.0 + (y_grid - mean) ** 2.0) / (2.0 * var))
    g = g / jnp.sum(g)
    w = (1.0 - g).reshape(_SS).astype(jnp.float32)
    return jnp.concatenate([w, jnp.zeros((_KROWS - _SS,), jnp.float32)])


def _pipelined_gather(table_hbm, idx_hbm, out_hbm, bufs, ks, n_points, pt,
                      base):
    idx_v = bufs[0:_NBUF]
    buf_v = bufs[_NBUF:2 * _NBUF]
    semi = bufs[2 * _NBUF:3 * _NBUF]
    semg = bufs[3 * _NBUF:4 * _NBUF]
    sems = bufs[4 * _NBUF:5 * _NBUF]

    def row(k):
        return pl.ds(k * n_points + base, pt)

    nk = len(ks)
    ld = [None] * nk
    gat = [None] * nk
    st = [None] * nk

    def fire_st(i):
        s = i % _NBUF
        gat[i].wait()
        st[i] = pltpu.async_copy(buf_v[s], out_hbm.at[row(ks[i])], sems[s])

    ld[0] = pltpu.async_copy(idx_hbm.at[row(ks[0])], idx_v[0], semi[0])
    for i in range(nk):
        s = i % _NBUF
        ld[i].wait()
        if i >= _NBUF:
            st[i - _NBUF].wait()
        gat[i] = pltpu.async_copy(table_hbm.at[idx_v[s]], buf_v[s], semg[s])
        if i + 1 < nk:
            if i + 1 >= _NBUF:
                fire_st(i + 1 - _NBUF)
            ld[i + 1] = pltpu.async_copy(
                idx_hbm.at[row(ks[i + 1])], idx_v[(i + 1) % _NBUF],
                semi[(i + 1) % _NBUF])
    for i in range(max(0, nk - _NBUF), nk):
        fire_st(i)
        st[i].wait()


def _sc_gather_range(idx_all, rng_pad, unproj, n_points):
    pt = n_points // _NW
    mesh = plsc.VectorSubcoreMesh(core_axis_name="c", subcore_axis_name="s",
                                  num_cores=_NC, num_subcores=_NS)
    scratch = ([pltpu.VMEM((pt,), jnp.int32) for _ in range(_NBUF)]
               + [pltpu.VMEM((pt,), jnp.float32) for _ in range(_NBUF)]
               + [pltpu.SemaphoreType.DMA for _ in range(3 * _NBUF)]
               + [pltpu.VMEM((pt,), jnp.float32), pltpu.SemaphoreType.DMA])

    @functools.partial(
        pl.kernel,
        out_type=jax.ShapeDtypeStruct((_KROWS * n_points,), jnp.float32),
        mesh=mesh,
        scratch_types=scratch,
    )
    def sc1(idx_hbm, rng_hbm, unp_hbm, grng_hbm, *bufs):
        unp_v = bufs[5 * _NBUF]
        semu = bufs[5 * _NBUF + 1]
        wid = lax.axis_index("s") * _NC + lax.axis_index("c")
        base = wid * pt
        pltpu.sync_copy(unp_hbm.at[pl.ds(base, pt)], unp_v)
        u1 = pltpu.async_copy(
            unp_v, grng_hbm.at[pl.ds(_CENTER * n_points + base, pt)], semu)
        u2 = pltpu.async_copy(
            unp_v, grng_hbm.at[pl.ds(_UNP_ROW * n_points + base, pt)], semu)
        ks = [k for k in range(_SS) if k != _CENTER]
        _pipelined_gather(rng_hbm, idx_hbm, grng_hbm, bufs[:5 * _NBUF], ks,
                          n_points, pt, base)
        u1.wait()
        u2.wait()

    return sc1(idx_all, rng_pad, unproj)


def _sc_gather_cls(sel_idx, cls_pad, n_points):
    pt = n_points // _NW
    mesh = plsc.VectorSubcoreMesh(core_axis_name="c", subcore_axis_name="s",
                                  num_cores=_NC, num_subcores=_NS)
    scratch = ([pltpu.VMEM((pt,), jnp.int32) for _ in range(_NBUF)]
               + [pltpu.VMEM((pt,), jnp.int32) for _ in range(_NBUF)]
               + [pltpu.SemaphoreType.DMA for _ in range(3 * _NBUF)])

    @functools.partial(
        pl.kernel,
        out_type=jax.ShapeDtypeStruct((_SROWS * n_points,), jnp.int32),
        mesh=mesh,
        scratch_types=scratch,
    )
    def sc2(selidx_hbm, cls_hbm, cls5_hbm, *bufs):
        wid = lax.axis_index("s") * _NC + lax.axis_index("c")
        base = wid * pt
        _pipelined_gather(cls_hbm, selidx_hbm, cls5_hbm, bufs,
                          list(range(_KNN)), n_points, pt, base)

    return sc2(sel_idx, cls_pad)


def _tc_select_body(sentinel, grng_ref, base_ref, w_ref, offs_ref, o_ref):
    g = grng_ref[...]
    w = w_ref[...]
    offs = offs_ref[...]
    b = g.shape[1]
    base = base_ref[...].reshape(1, b)
    r = g[_UNP_ROW:_UNP_ROW + 1, :]
    rows = lax.broadcasted_iota(jnp.int32, (_KROWS, b), 0)
    d = jnp.abs(g - r) * w
    d = jnp.where(rows < _SS, d, jnp.inf)

    sel = []
    for _ in range(_KNN):
        m = jnp.min(d, axis=0, keepdims=True)
        ki = jnp.min(jnp.where(d == m, rows, _KROWS), axis=0, keepdims=True)
        hit = rows == ki
        off = jnp.max(jnp.where(hit, offs, -1), axis=0, keepdims=True)
        flat = jnp.where(m > _CUTOFF, sentinel, base + off)
        sel.append(flat)
        d = jnp.where(hit, jnp.inf, d)
    zero = jnp.zeros_like(sel[0])
    o_ref[...] = jnp.concatenate(sel + [zero] * (_SROWS - _KNN), axis=0)


def _tc_select(g_rng, base3, w_col, offs_col, sentinel, n_points, block=2048):
    nb = n_points // block
    return pl.pallas_call(
        functools.partial(_tc_select_body, sentinel),
        grid=(nb,),
        in_specs=[
            pl.BlockSpec((_KROWS, block), lambda i: (0, i)),
            pl.BlockSpec((1, 1, block), lambda i: (i, 0, 0)),
            pl.BlockSpec((_KROWS, 1), lambda i: (0, 0)),
            pl.BlockSpec((_KROWS, 1), lambda i: (0, 0)),
        ],
        out_specs=pl.BlockSpec((_SROWS, block), lambda i: (0, i)),
        out_shape=jax.ShapeDtypeStruct((_SROWS, n_points), jnp.int32),
    )(g_rng, base3, w_col, offs_col)


def _tc_vote_body(cls_ref, o_ref):
    cl = cls_ref[...]
    b = cl.shape[1]
    sel = [cl[i:i + 1, :] for i in range(_KNN)]
    ones = jnp.ones_like(sel[0])
    cnt = [ones] * _KNN
    for i in range(_KNN):
        for j in range(i + 1, _KNN):
            e = (sel[i] == sel[j]).astype(jnp.int32)
            cnt[i] = cnt[i] + e
            cnt[j] = cnt[j] + e
    neg = jnp.full_like(ones, -1000)
    key = neg
    for i in range(_KNN):
        c = sel[i]
        valid = (c >= 1) & (c < _NCLS)
        key = jnp.maximum(key, jnp.where(valid, cnt[i] * 32 - c, neg))
    best = jnp.where(key == -1000, 1, 32 - (key & 31))
    o_ref[...] = best.reshape(1, 1, b)


def _tc_vote(cls5, n_points, block=2048):
    nb = n_points // block
    return pl.pallas_call(
        _tc_vote_body,
        grid=(nb,),
        in_specs=[pl.BlockSpec((_SROWS, block), lambda i: (0, i))],
        out_specs=pl.BlockSpec((1, 1, block), lambda i: (i, 0, 0)),
        out_shape=jax.ShapeDtypeStruct((nb, 1, block), jnp.int32),
    )(cls5)


def kernel(proj_range, unproj_range, proj_argmax, px, py):
    h, w = proj_range.shape
    p = unproj_range.shape[0]
    pad = (_S - 1) // 2
    wp = w + 2 * pad
    rng_pad = jnp.pad(proj_range, pad).reshape(-1)
    npix = rng_pad.shape[0]
    cls_pad = jnp.concatenate([
        jnp.pad(proj_argmax, pad).reshape(-1),
        jnp.full((8,), _NCLS, jnp.int32)])
    sentinel = npix
    base = py * wp + px
    offs = [dy * wp + dx for dy in range(_S) for dx in range(_S)]
    idx_all = (jnp.array(offs, jnp.int32)[:, None] + base[None, :])
    g_rng = _sc_gather_range(idx_all.reshape(-1), rng_pad, unproj_range, p)
    g_rng = g_rng.reshape(_KROWS, p)
    w_col = _inv_gauss_weights().reshape(_KROWS, 1)
    offs_col = jnp.array(offs + [0] * (_KROWS - _SS),
                         jnp.int32).reshape(_KROWS, 1)
    base3 = base.reshape(p // 2048, 1, 2048)
    sel_idx = _tc_select(g_rng, base3, w_col, offs_col, sentinel, p)
    cls5 = _sc_gather_cls(sel_idx.reshape(-1), cls_pad, p)
    out3 = _tc_vote(cls5.reshape(_SROWS, p), p)
    return out3.reshape(p)


# speedup vs baseline: 2.2733x; 1.1210x over previous
import functools
import math

import jax
import jax.numpy as jnp
from jax import lax
from jax.experimental import pallas as pl
from jax.experimental.pallas import tpu as pltpu
from jax.experimental.pallas import tpu_sc as plsc

_KNN = 5
_S = 5
_SS = _S * _S
_CENTER = (_SS - 1) // 2
_SIGMA = 1.0
_CUTOFF = 1.0
_NCLS = 20
_KROWS = 32
_UNP_ROW = 25
_SROWS = 8

_NC, _NS = 2, 16
_NW = _NC * _NS
_NBUF = 3


def _inv_gauss_weights():
    x = jnp.arange(_S)
    x_grid = jnp.tile(x, _S).reshape(_S, _S)
    y_grid = x_grid.T
    mean = (_S - 1) / 2.0
    var = _SIGMA ** 2.0
    g = (1.0 / (2.0 * math.pi * var)) * jnp.exp(
        -((x_grid - mean) ** 2.0 + (y_grid - mean) ** 2.0) / (2.0 * var))
    g = g / jnp.sum(g)
    w = (1.0 - g).reshape(_SS).astype(jnp.float32)
    return jnp.concatenate([w, jnp.zeros((_KROWS - _SS,), jnp.float32)])


def _pipelined_gather(table_hbm, idx_hbm, out_hbm, bufs, ks, n_points, pt,
                      base):
    idx_v = bufs[0:_NBUF]
    buf_v = bufs[_NBUF:2 * _NBUF]
    semi = bufs[2 * _NBUF:3 * _NBUF]
    semg = bufs[3 * _NBUF:4 * _NBUF]
    sems = bufs[4 * _NBUF:5 * _NBUF]

    def row(k):
        return pl.ds(k * n_points + base, pt)

    nk = len(ks)
    ld = [None] * nk
    gat = [None] * nk
    st = [None] * nk

    def fire_st(i):
        s = i % _NBUF
        gat[i].wait()
        st[i] = pltpu.async_copy(buf_v[s], out_hbm.at[row(ks[i])], sems[s])

    ld[0] = pltpu.async_copy(idx_hbm.at[row(ks[0])], idx_v[0], semi[0])
    for i in range(nk):
        s = i % _NBUF
        ld[i].wait()
        if i >= _NBUF:
            st[i - _NBUF].wait()
        gat[i] = pltpu.async_copy(table_hbm.at[idx_v[s]], buf_v[s], semg[s])
        if i + 1 < nk:
            if i + 1 >= _NBUF:
                fire_st(i + 1 - _NBUF)
            ld[i + 1] = pltpu.async_copy(
                idx_hbm.at[row(ks[i + 1])], idx_v[(i + 1) % _NBUF],
                semi[(i + 1) % _NBUF])
    for i in range(max(0, nk - _NBUF), nk):
        fire_st(i)
        st[i].wait()


def _sc_gather_range(idx_all, rng_pad, unproj, n_points):
    pt = n_points // _NW
    mesh = plsc.VectorSubcoreMesh(core_axis_name="c", subcore_axis_name="s",
                                  num_cores=_NC, num_subcores=_NS)
    scratch = ([pltpu.VMEM((pt,), jnp.int32) for _ in range(_NBUF)]
               + [pltpu.VMEM((pt,), jnp.float32) for _ in range(_NBUF)]
               + [pltpu.SemaphoreType.DMA for _ in range(3 * _NBUF)]
               + [pltpu.VMEM((pt,), jnp.float32), pltpu.SemaphoreType.DMA])

    @functools.partial(
        pl.kernel,
        out_type=jax.ShapeDtypeStruct((_KROWS * n_points,), jnp.float32),
        mesh=mesh,
        scratch_types=scratch,
    )
    def sc1(idx_hbm, rng_hbm, unp_hbm, grng_hbm, *bufs):
        unp_v = bufs[5 * _NBUF]
        semu = bufs[5 * _NBUF + 1]
        wid = lax.axis_index("s") * _NC + lax.axis_index("c")
        base = wid * pt
        pltpu.sync_copy(unp_hbm.at[pl.ds(base, pt)], unp_v)
        u1 = pltpu.async_copy(
            unp_v, grng_hbm.at[pl.ds(_CENTER * n_points + base, pt)], semu)
        u2 = pltpu.async_copy(
            unp_v, grng_hbm.at[pl.ds(_UNP_ROW * n_points + base, pt)], semu)
        ks = [k for k in range(_SS) if k != _CENTER]
        _pipelined_gather(rng_hbm, idx_hbm, grng_hbm, bufs[:5 * _NBUF], ks,
                          n_points, pt, base)
        u1.wait()
        u2.wait()

    return sc1(idx_all, rng_pad, unproj)


def _sc_cls_vote(sel_idx, cls_pad, n_points):
    pt = n_points // _NW
    mesh = plsc.VectorSubcoreMesh(core_axis_name="c", subcore_axis_name="s",
                                  num_cores=_NC, num_subcores=_NS)
    scratch = ([pltpu.VMEM((pt,), jnp.int32) for _ in range(_KNN)]
               + [pltpu.VMEM((pt,), jnp.int32) for _ in range(_KNN)]
               + [pltpu.VMEM((pt,), jnp.int32)]
               + [pltpu.SemaphoreType.DMA for _ in range(2 * _KNN + 1)])

    @functools.partial(
        pl.kernel,
        out_type=jax.ShapeDtypeStruct((n_points,), jnp.int32),
        mesh=mesh,
        scratch_types=scratch,
        compiler_params=pltpu.CompilerParams(needs_layout_passes=False),
    )
    def sc2(selidx_hbm, cls_hbm, out_hbm, *bufs):
        idx_v = bufs[0:_KNN]
        cls_v = bufs[_KNN:2 * _KNN]
        out_v = bufs[2 * _KNN]
        semi = bufs[2 * _KNN + 1:3 * _KNN + 1]
        semg = bufs[3 * _KNN + 1:4 * _KNN + 1]
        semo = bufs[4 * _KNN + 1]
        wid = lax.axis_index("s") * _NC + lax.axis_index("c")
        base = wid * pt
        ld = [pltpu.async_copy(
            selidx_hbm.at[pl.ds(j * n_points + base, pt)], idx_v[j], semi[j])
            for j in range(_KNN)]
        gat = []
        for j in range(_KNN):
            ld[j].wait()
            gat.append(
                pltpu.async_copy(cls_hbm.at[idx_v[j]], cls_v[j], semg[j]))
        for g in gat:
            g.wait()

        def vote(gi, _):
            off = gi * 16
            sel = [cls_v[j][pl.ds(off, 16)] for j in range(_KNN)]
            ones = jnp.ones((16,), jnp.int32)
            cnt = [ones] * _KNN
            for i in range(_KNN):
                for j in range(i + 1, _KNN):
                    e = (sel[i] == sel[j]).astype(jnp.int32)
                    cnt[i] = cnt[i] + e
                    cnt[j] = cnt[j] + e
            neg = jnp.full((16,), -1000, jnp.int32)
            key = neg
            for i in range(_KNN):
                c = sel[i]
                valid = (c >= 1) & (c < _NCLS)
                key = jnp.maximum(key, jnp.where(valid, cnt[i] * 32 - c, neg))
            best = jnp.where(key == -1000, 1, 32 - (key & 31))
            out_v[pl.ds(off, 16)] = best
            return 0

        lax.fori_loop(0, pt // 16, vote, 0)
        pltpu.async_copy(out_v, out_hbm.at[pl.ds(base, pt)], semo).wait()

    return sc2(sel_idx, cls_pad)


def _tc_select_body(sentinel, grng_ref, base_ref, w_ref, offs_ref, o_ref):
    g = grng_ref[...]
    w = w_ref[...]
    offs = offs_ref[...]
    b = g.shape[1]
    base = base_ref[...].reshape(1, b)
    r = g[_UNP_ROW:_UNP_ROW + 1, :]
    rows = lax.broadcasted_iota(jnp.int32, (_KROWS, b), 0)
    d = jnp.abs(g - r) * w
    d = jnp.where(rows < _SS, d, jnp.inf)

    sel = []
    for _ in range(_KNN):
        m = jnp.min(d, axis=0, keepdims=True)
        ki = jnp.min(jnp.where(d == m, rows, _KROWS), axis=0, keepdims=True)
        hit = rows == ki
        off = jnp.max(jnp.where(hit, offs, -1), axis=0, keepdims=True)
        flat = jnp.where(m > _CUTOFF, sentinel, base + off)
        sel.append(flat)
        d = jnp.where(hit, jnp.inf, d)
    zero = jnp.zeros_like(sel[0])
    o_ref[...] = jnp.concatenate(sel + [zero] * (_SROWS - _KNN), axis=0)


def _tc_select(g_rng, base3, w_col, offs_col, sentinel, n_points, block=2048):
    nb = n_points // block
    return pl.pallas_call(
        functools.partial(_tc_select_body, sentinel),
        grid=(nb,),
        in_specs=[
            pl.BlockSpec((_KROWS, block), lambda i: (0, i)),
            pl.BlockSpec((1, 1, block), lambda i: (i, 0, 0)),
            pl.BlockSpec((_KROWS, 1), lambda i: (0, 0)),
            pl.BlockSpec((_KROWS, 1), lambda i: (0, 0)),
        ],
        out_specs=pl.BlockSpec((_SROWS, block), lambda i: (0, i)),
        out_shape=jax.ShapeDtypeStruct((_SROWS, n_points), jnp.int32),
    )(g_rng, base3, w_col, offs_col)


def _tc_vote_body(cls_ref, o_ref):
    cl = cls_ref[...]
    b = cl.shape[1]
    sel = [cl[i:i + 1, :] for i in range(_KNN)]
    ones = jnp.ones_like(sel[0])
    cnt = [ones] * _KNN
    for i in range(_KNN):
        for j in range(i + 1, _KNN):
            e = (sel[i] == sel[j]).astype(jnp.int32)
            cnt[i] = cnt[i] + e
            cnt[j] = cnt[j] + e
    neg = jnp.full_like(ones, -1000)
    key = neg
    for i in range(_KNN):
        c = sel[i]
        valid = (c >= 1) & (c < _NCLS)
        key = jnp.maximum(key, jnp.where(valid, cnt[i] * 32 - c, neg))
    best = jnp.where(key == -1000, 1, 32 - (key & 31))
    o_ref[...] = best.reshape(1, 1, b)


def _tc_vote(cls5, n_points, block=2048):
    nb = n_points // block
    return pl.pallas_call(
        _tc_vote_body,
        grid=(nb,),
        in_specs=[pl.BlockSpec((_SROWS, block), lambda i: (0, i))],
        out_specs=pl.BlockSpec((1, 1, block), lambda i: (i, 0, 0)),
        out_shape=jax.ShapeDtypeStruct((nb, 1, block), jnp.int32),
    )(cls5)


def kernel(proj_range, unproj_range, proj_argmax, px, py):
    h, w = proj_range.shape
    p = unproj_range.shape[0]
    pad = (_S - 1) // 2
    wp = w + 2 * pad
    rng_pad = jnp.pad(proj_range, pad).reshape(-1)
    npix = rng_pad.shape[0]
    cls_pad = jnp.concatenate([
        jnp.pad(proj_argmax, pad).reshape(-1),
        jnp.full((8,), _NCLS, jnp.int32)])
    sentinel = npix
    base = py * wp + px
    offs = [dy * wp + dx for dy in range(_S) for dx in range(_S)]
    idx_all = (jnp.array(offs, jnp.int32)[:, None] + base[None, :])
    g_rng = _sc_gather_range(idx_all.reshape(-1), rng_pad, unproj_range, p)
    g_rng = g_rng.reshape(_KROWS, p)
    w_col = _inv_gauss_weights().reshape(_KROWS, 1)
    offs_col = jnp.array(offs + [0] * (_KROWS - _SS),
                         jnp.int32).reshape(_KROWS, 1)
    base3 = base.reshape(p // 2048, 1, 2048)
    sel_idx = _tc_select(g_rng, base3, w_col, offs_col, sentinel, p)
    return _sc_cls_vote(sel_idx.reshape(-1), cls_pad, p)


# speedup vs baseline: 2.7006x; 1.1880x over previous
import functools
import math

import jax
import jax.numpy as jnp
from jax import lax
from jax.experimental import pallas as pl
from jax.experimental.pallas import tpu as pltpu
from jax.experimental.pallas import tpu_sc as plsc

_KNN = 5
_S = 5
_SS = _S * _S
_CENTER = (_SS - 1) // 2
_SIGMA = 1.0
_CUTOFF = 1.0
_NCLS = 20
_KROWS = 32
_UNP_ROW = 25
_SROWS = 8

_NC, _NS = 2, 16
_NW = _NC * _NS
_NBUF = 3


def _inv_gauss_weights():
    x = jnp.arange(_S)
    x_grid = jnp.tile(x, _S).reshape(_S, _S)
    y_grid = x_grid.T
    mean = (_S - 1) / 2.0
    var = _SIGMA ** 2.0
    g = (1.0 / (2.0 * math.pi * var)) * jnp.exp(
        -((x_grid - mean) ** 2.0 + (y_grid - mean) ** 2.0) / (2.0 * var))
    g = g / jnp.sum(g)
    w = (1.0 - g).reshape(_SS).astype(jnp.float32)
    return jnp.concatenate([w, jnp.zeros((_KROWS - _SS,), jnp.float32)])


def _pipelined_gather(table_hbm, idx_hbm, out_hbm, bufs, ks, n_points, pt,
                      base):
    idx_v = bufs[0:_NBUF]
    buf_v = bufs[_NBUF:2 * _NBUF]
    semi = bufs[2 * _NBUF:3 * _NBUF]
    semg = bufs[3 * _NBUF:4 * _NBUF]
    sems = bufs[4 * _NBUF:5 * _NBUF]

    def row(k):
        return pl.ds(k * n_points + base, pt)

    nk = len(ks)
    ld = [None] * nk
    gat = [None] * nk
    st = [None] * nk

    def fire_st(i):
        s = i % _NBUF
        gat[i].wait()
        st[i] = pltpu.async_copy(buf_v[s], out_hbm.at[row(ks[i])], sems[s])

    ld[0] = pltpu.async_copy(idx_hbm.at[row(ks[0])], idx_v[0], semi[0])
    for i in range(nk):
        s = i % _NBUF
        ld[i].wait()
        if i >= _NBUF:
            st[i - _NBUF].wait()
        gat[i] = pltpu.async_copy(table_hbm.at[idx_v[s]], buf_v[s], semg[s])
        if i + 1 < nk:
            if i + 1 >= _NBUF:
                fire_st(i + 1 - _NBUF)
            ld[i + 1] = pltpu.async_copy(
                idx_hbm.at[row(ks[i + 1])], idx_v[(i + 1) % _NBUF],
                semi[(i + 1) % _NBUF])
    for i in range(max(0, nk - _NBUF), nk):
        fire_st(i)
        st[i].wait()


def _sc_gather_range(idx_all, rng_pad, unproj, n_points):
    pt = n_points // _NW
    mesh = plsc.VectorSubcoreMesh(core_axis_name="c", subcore_axis_name="s",
                                  num_cores=_NC, num_subcores=_NS)
    scratch = ([pltpu.VMEM((pt,), jnp.int32) for _ in range(_NBUF)]
               + [pltpu.VMEM((pt,), jnp.float32) for _ in range(_NBUF)]
               + [pltpu.SemaphoreType.DMA for _ in range(3 * _NBUF)]
               + [pltpu.VMEM((pt,), jnp.float32), pltpu.SemaphoreType.DMA])

    @functools.partial(
        pl.kernel,
        out_type=jax.ShapeDtypeStruct((_KROWS * n_points,), jnp.float32),
        mesh=mesh,
        scratch_types=scratch,
    )
    def sc1(idx_hbm, rng_hbm, unp_hbm, grng_hbm, *bufs):
        unp_v = bufs[5 * _NBUF]
        semu = bufs[5 * _NBUF + 1]
        wid = lax.axis_index("s") * _NC + lax.axis_index("c")
        base = wid * pt
        pltpu.sync_copy(unp_hbm.at[pl.ds(base, pt)], unp_v)
        u1 = pltpu.async_copy(
            unp_v, grng_hbm.at[pl.ds(_CENTER * n_points + base, pt)], semu)
        u2 = pltpu.async_copy(
            unp_v, grng_hbm.at[pl.ds(_UNP_ROW * n_points + base, pt)], semu)
        ks = [k for k in range(_SS) if k != _CENTER]
        _pipelined_gather(rng_hbm, idx_hbm, grng_hbm, bufs[:5 * _NBUF], ks,
                          n_points, pt, base)
        u1.wait()
        u2.wait()

    return sc1(idx_all, rng_pad, unproj)


def _sc_cls_vote(sel_idx, cls_pad, n_points):
    pt = n_points // _NW
    mesh = plsc.VectorSubcoreMesh(core_axis_name="c", subcore_axis_name="s",
                                  num_cores=_NC, num_subcores=_NS)
    scratch = ([pltpu.VMEM((pt,), jnp.int32) for _ in range(_KNN)]
               + [pltpu.VMEM((pt,), jnp.int32) for _ in range(_KNN)]
               + [pltpu.VMEM((pt,), jnp.int32)]
               + [pltpu.SemaphoreType.DMA for _ in range(2 * _KNN + 1)])

    @functools.partial(
        pl.kernel,
        out_type=jax.ShapeDtypeStruct((n_points,), jnp.int32),
        mesh=mesh,
        scratch_types=scratch,
        compiler_params=pltpu.CompilerParams(needs_layout_passes=False),
    )
    def sc2(selidx_hbm, cls_hbm, out_hbm, *bufs):
        idx_v = bufs[0:_KNN]
        cls_v = bufs[_KNN:2 * _KNN]
        out_v = bufs[2 * _KNN]
        semi = bufs[2 * _KNN + 1:3 * _KNN + 1]
        semg = bufs[3 * _KNN + 1:4 * _KNN + 1]
        semo = bufs[4 * _KNN + 1]
        wid = lax.axis_index("s") * _NC + lax.axis_index("c")
        base = wid * pt
        ld = [pltpu.async_copy(
            selidx_hbm.at[pl.ds(j * n_points + base, pt)], idx_v[j], semi[j])
            for j in range(_KNN)]
        gat = []
        for j in range(_KNN):
            ld[j].wait()
            gat.append(
                pltpu.async_copy(cls_hbm.at[idx_v[j]], cls_v[j], semg[j]))
        for g in gat:
            g.wait()

        def vote(gi, _):
            off = gi * 16
            sel = [cls_v[j][pl.ds(off, 16)] for j in range(_KNN)]
            ones = jnp.ones((16,), jnp.int32)
            cnt = [ones] * _KNN
            for i in range(_KNN):
                for j in range(i + 1, _KNN):
                    e = (sel[i] == sel[j]).astype(jnp.int32)
                    cnt[i] = cnt[i] + e
                    cnt[j] = cnt[j] + e
            neg = jnp.full((16,), -1000, jnp.int32)
            key = neg
            for i in range(_KNN):
                c = sel[i]
                valid = (c >= 1) & (c < _NCLS)
                key = jnp.maximum(key, jnp.where(valid, cnt[i] * 32 - c, neg))
            best = jnp.where(key == -1000, 1, 32 - (key & 31))
            out_v[pl.ds(off, 16)] = best
            return 0

        lax.fori_loop(0, pt // 16, vote, 0)
        pltpu.async_copy(out_v, out_hbm.at[pl.ds(base, pt)], semo).wait()

    return sc2(sel_idx, cls_pad)


def _tc_select_body(sentinel, grng_ref, base_ref, w_ref, offs_ref, o_ref):
    g = grng_ref[...]
    w = w_ref[...]
    offs = offs_ref[...]
    b = g.shape[1]
    base = base_ref[...].reshape(1, b)
    r = g[_UNP_ROW:_UNP_ROW + 1, :]
    rows = lax.broadcasted_iota(jnp.int32, (_KROWS, b), 0)
    d = jnp.abs(g - r) * w
    d = jnp.where(rows < _SS, d, jnp.inf)

    sel = []
    for _ in range(_KNN):
        m = jnp.min(d, axis=0, keepdims=True)
        ki = jnp.min(jnp.where(d == m, rows, _KROWS), axis=0, keepdims=True)
        hit = rows == ki
        off = jnp.max(jnp.where(hit, offs, -1), axis=0, keepdims=True)
        flat = jnp.where(m > _CUTOFF, sentinel, base + off)
        sel.append(flat)
        d = jnp.where(hit, jnp.inf, d)
    zero = jnp.zeros_like(sel[0])
    o_ref[...] = jnp.concatenate(sel + [zero] * (_SROWS - _KNN), axis=0)


def _tc_select(g_rng, base3, w_col, offs_col, sentinel, n_points, block=2048):
    nb = n_points // block
    return pl.pallas_call(
        functools.partial(_tc_select_body, sentinel),
        grid=(nb,),
        in_specs=[
            pl.BlockSpec((_KROWS, block), lambda i: (0, i)),
            pl.BlockSpec((1, 1, block), lambda i: (i, 0, 0)),
            pl.BlockSpec((_KROWS, 1), lambda i: (0, 0)),
            pl.BlockSpec((_KROWS, 1), lambda i: (0, 0)),
        ],
        out_specs=pl.BlockSpec((_SROWS, block), lambda i: (0, i)),
        out_shape=jax.ShapeDtypeStruct((_SROWS, n_points), jnp.int32),
    )(g_rng, base3, w_col, offs_col)


def _tc_vote_body(cls_ref, o_ref):
    cl = cls_ref[...]
    b = cl.shape[1]
    sel = [cl[i:i + 1, :] for i in range(_KNN)]
    ones = jnp.ones_like(sel[0])
    cnt = [ones] * _KNN
    for i in range(_KNN):
        for j in range(i + 1, _KNN):
            e = (sel[i] == sel[j]).astype(jnp.int32)
            cnt[i] = cnt[i] + e
            cnt[j] = cnt[j] + e
    neg = jnp.full_like(ones, -1000)
    key = neg
    for i in range(_KNN):
        c = sel[i]
        valid = (c >= 1) & (c < _NCLS)
        key = jnp.maximum(key, jnp.where(valid, cnt[i] * 32 - c, neg))
    best = jnp.where(key == -1000, 1, 32 - (key & 31))
    o_ref[...] = best.reshape(1, 1, b)


def _tc_vote(cls5, n_points, block=2048):
    nb = n_points // block
    return pl.pallas_call(
        _tc_vote_body,
        grid=(nb,),
        in_specs=[pl.BlockSpec((_SROWS, block), lambda i: (0, i))],
        out_specs=pl.BlockSpec((1, 1, block), lambda i: (i, 0, 0)),
        out_shape=jax.ShapeDtypeStruct((nb, 1, block), jnp.int32),
    )(cls5)


def kernel(proj_range, unproj_range, proj_argmax, px, py):
    h, w = proj_range.shape
    p = unproj_range.shape[0]
    pad = (_S - 1) // 2
    wp = w + 2 * pad
    rng_pad = jnp.pad(proj_range, pad).reshape(-1)
    npix = rng_pad.shape[0]
    cls_pad = jnp.concatenate([
        jnp.pad(proj_argmax, pad).reshape(-1),
        jnp.full((8,), _NCLS, jnp.int32)])
    sentinel = npix
    base = py * wp + px
    offs = [dy * wp + dx for dy in range(_S) for dx in range(_S)]
    offs_arr = jnp.array(offs, jnp.int32)
    w_col = _inv_gauss_weights().reshape(_KROWS, 1)
    offs_col = jnp.array(offs + [0] * (_KROWS - _SS),
                         jnp.int32).reshape(_KROWS, 1)

    nh = 2
    ph = p // nh
    outs = []
    for hh in range(nh):
        base_h = lax.slice(base, (hh * ph,), ((hh + 1) * ph,))
        unp_h = lax.slice(unproj_range, (hh * ph,), ((hh + 1) * ph,))
        idx_h = offs_arr[:, None] + base_h[None, :]
        g_rng = _sc_gather_range(idx_h.reshape(-1), rng_pad, unp_h, ph)
        g_rng = g_rng.reshape(_KROWS, ph)
        base3 = base_h.reshape(ph // 2048, 1, 2048)
        sel_idx = _tc_select(g_rng, base3, w_col, offs_col, sentinel, ph)
        outs.append(_sc_cls_vote(sel_idx.reshape(-1), cls_pad, ph))
    return jnp.concatenate(outs)
